# baseline (speedup 1.0000x reference)
import functools

import jax
import jax.numpy as jnp
from jax import lax
from jax.experimental import pallas as pl
from jax.experimental.pallas import tpu as pltpu
from jax.experimental.pallas import tpu_sc as plsc

N_NODES = 10000
N_EDGES = 320000
D = 128

NC = 2
NS = 16
NW = NC * NS

CHUNK = 128
NQ = N_EDGES // CHUNK
NQT = NQ // NW
NBUF = 2
N_IDX = 88
NQ_PAD = 2504
ROWS_MOST = 640
ROWS_LAST = N_NODES - 15 * ROWS_MOST
ZROWS = 8

_SC_MESH = plsc.VectorSubcoreMesh(
    core_axis_name="c", subcore_axis_name="s", num_cores=NC, num_subcores=NS)


@functools.partial(
    pl.kernel,
    out_type=jax.ShapeDtypeStruct((NC, N_NODES, D), jnp.float32),
    mesh=_SC_MESH,
    scratch_types=[
        pltpu.VMEM_SHARED((N_NODES, D), jnp.float32),
        pltpu.VMEM((N_IDX, CHUNK), jnp.int32),
        pltpu.VMEM((NBUF, CHUNK, D), jnp.float32),
        pltpu.VMEM((ZROWS, D), jnp.float32),
        pltpu.SemaphoreType.DMA,
        pltpu.SemaphoreType.DMA,
        pltpu.SemaphoreType.DMA,
        pltpu.SemaphoreType.DMA,
        pltpu.SemaphoreType.DMA,
    ],
)
def _sc_scatter_add(e_hbm, dst_hbm, out_hbm, acc, idx_v, ebuf, zbuf,
                    idx_sem, fill_sem0, fill_sem1, scat_sem0, scat_sem1):
    c = lax.axis_index("c")
    s = lax.axis_index("s")
    fill_sems = (fill_sem0, fill_sem1)
    scat_sems = (scat_sem0, scat_sem1)

    wid = c * NS + s
    qs = wid * NQT + jnp.maximum(wid - 28, 0)
    a = 8 * (qs // 8)
    r = qs - a

    def _fill(k, b):
        pltpu.async_copy(e_hbm.at[pl.ds((qs + k) * CHUNK, CHUNK)],
                         ebuf.at[b], fill_sems[b])

    pltpu.async_copy(dst_hbm.at[pl.ds(a, N_IDX)], idx_v, idx_sem)
    _fill(0, 0)
    _fill(1, 1)

    def _zrow(i, _):
        def _zcol(j, _):
            zbuf[i, pl.ds(j * 16, 16)] = jnp.zeros((16,), jnp.float32)
            return 0
        return lax.fori_loop(0, D // 16, _zcol, 0)
    lax.fori_loop(0, ZROWS, _zrow, 0)

    def _zcp(t, _):
        pltpu.sync_copy(zbuf, acc.at[pl.ds(s * ROWS_MOST + t * ZROWS, ZROWS)])
        return 0

    @pl.when(s < 15)
    def _():
        lax.fori_loop(0, ROWS_MOST // ZROWS, _zcp, 0)

    @pl.when(s == 15)
    def _():
        lax.fori_loop(0, ROWS_LAST // ZROWS, _zcp, 0)

    pltpu.make_async_copy(dst_hbm.at[pl.ds(a, N_IDX)], idx_v, idx_sem).wait()
    plsc.subcore_barrier()

    def _wait_fill(b):
        pltpu.make_async_copy(e_hbm.at[pl.ds(0, CHUNK)],
                              ebuf.at[b], fill_sems[b]).wait()

    def _drain_scat(b):
        pltpu.make_async_copy(e_hbm.at[pl.ds(0, CHUNK)],
                              ebuf.at[b], scat_sems[b]).wait()

    def _step(k, b, do_fill=True):
        _wait_fill(b)
        pltpu.async_copy(ebuf.at[b], acc.at[idx_v.at[r + k]], scat_sems[b],
                         add=True)
        _drain_scat(b)
        if do_fill:
            _fill(k + NBUF, b)

    def _pair(t, _):
        _step(2 * t, 0)
        _step(2 * t + 1, 1)
        return 0
    lax.fori_loop(0, (NQT - 2) // 2, _pair, 0)

    _step(NQT - 2, 0, do_fill=False)

    @pl.when(wid >= 28)
    def _():
        _fill(NQT, 0)

    _step(NQT - 1, 1, do_fill=False)

    @pl.when(wid >= 28)
    def _():
        _step(NQT, 0, do_fill=False)
    plsc.subcore_barrier()

    @pl.when(s < 15)
    def _():
        pltpu.sync_copy(acc.at[pl.ds(s * ROWS_MOST, ROWS_MOST)],
                        out_hbm.at[c, pl.ds(s * ROWS_MOST, ROWS_MOST)])

    @pl.when(s == 15)
    def _():
        pltpu.sync_copy(acc.at[pl.ds(15 * ROWS_MOST, ROWS_LAST)],
                        out_hbm.at[c, pl.ds(15 * ROWS_MOST, ROWS_LAST)])


_ROWS_BLK = 1000


def _matmul_t(x, w):
    return lax.dot_general(x, w, (((1,), (1,)), ((), ())),
                           preferred_element_type=jnp.float32)


def _mlp_body(p0_ref, p1_ref, v_ref, w1_ref, b1_ref, w2_ref, b2_ref, o_ref):
    x = p0_ref[...] + p1_ref[...]
    h = _matmul_t(x, w1_ref[...]) + b1_ref[...]
    sp = jnp.maximum(h, 0.0) + jnp.log1p(jnp.exp(-jnp.abs(h)))
    sp = sp - jnp.log(jnp.float32(2.0))
    o_ref[...] = v_ref[...] + b2_ref[...] + _matmul_t(sp, w2_ref[...])


def _mlp(p0, p1, v, W1, b1, W2, b2):
    grid = (N_NODES // _ROWS_BLK,)
    row_spec = pl.BlockSpec((_ROWS_BLK, D), lambda i: (i, 0))
    w_spec = pl.BlockSpec((D, D), lambda i: (0, 0))
    b_spec = pl.BlockSpec((1, D), lambda i: (0, 0))
    return pl.pallas_call(
        _mlp_body,
        grid=grid,
        in_specs=[row_spec, row_spec, row_spec, w_spec, b_spec, w_spec, b_spec],
        out_specs=row_spec,
        out_shape=jax.ShapeDtypeStruct((N_NODES, D), jnp.float32),
    )(p0, p1, v, W1, b1, W2, b2)


def kernel(v, e, edge_index, W1, b1, W2, b2):
    dst = edge_index[1].reshape(NQ, CHUNK)
    dst = jnp.pad(dst, ((0, NQ_PAD - NQ), (0, 0)))
    partials = _sc_scatter_add(e, dst)
    return _mlp(partials[0], partials[1], v,
                W1, b1.reshape(1, D), W2, b2.reshape(1, D))
